# baseline (speedup 1.0000x reference)
import functools

import jax
import jax.numpy as jnp
from jax import lax
from jax.experimental import pallas as pl
from jax.experimental.pallas import tpu as pltpu

E = 8
TOPK = 2
T = 2048
D = 1024
F = 1024
A = T * TOPK
BM = 256
NB = A // BM
NT = NB + E - 1


def _routing_plan(flat_e):
    onehot = (flat_e[:, None] == jnp.arange(E, dtype=jnp.int32)[None, :])
    csum = jnp.cumsum(onehot.astype(jnp.int32), axis=0)
    counts = csum[-1]
    offsets = jnp.concatenate(
        [jnp.zeros((1,), jnp.int32), jnp.cumsum(counts, dtype=jnp.int32)])
    rank = jnp.sum(onehot * csum, axis=1) - 1
    pos = offsets[flat_e] + rank

    start, end = offsets[:E], offsets[1:]
    nonempty = end > start
    first_blk = start // BM
    nb_e = jnp.where(nonempty, (end - 1) // BM - first_blk + 1, 0)
    cum_t = jnp.concatenate(
        [jnp.zeros((1,), jnp.int32), jnp.cumsum(nb_e, dtype=jnp.int32)])
    total = cum_t[E]
    slot = jnp.arange(NT, dtype=jnp.int32)
    valid = slot < total
    e_of = jnp.clip(
        jnp.searchsorted(cum_t, slot, side="right").astype(jnp.int32) - 1,
        0, E - 1)
    m_of = jnp.where(valid, first_blk[e_of] + (slot - cum_t[e_of]), NB - 1)
    e_t = jnp.where(valid, e_of, 0)

    ep = jnp.searchsorted(offsets, pos, side="right").astype(jnp.int32) - 1
    tile_of = cum_t[ep] + (pos // BM - first_blk[ep])
    gidx = tile_of * BM + pos % BM
    return pos, gidx, m_of, e_t


def _ffn_tile(m_r, e_r, x_ref, wu_ref, wd_ref, o_ref):
    h = jnp.dot(x_ref[...].astype(jnp.bfloat16), wu_ref[0],
                preferred_element_type=jnp.float32)
    a = jnp.maximum(h, 0.0)
    a2 = (a * a).astype(jnp.bfloat16)
    o_ref[0] = jnp.dot(a2, wd_ref[0], preferred_element_type=jnp.float32)


def _grouped_ffn(x_sorted, w_up, w_down, tile_m, tile_e):
    grid_spec = pltpu.PrefetchScalarGridSpec(
        num_scalar_prefetch=2,
        grid=(NT,),
        in_specs=[
            pl.BlockSpec((BM, D), lambda i, m, e: (m[i], 0)),
            pl.BlockSpec((1, D, F), lambda i, m, e: (e[i], 0, 0)),
            pl.BlockSpec((1, F, D), lambda i, m, e: (e[i], 0, 0)),
        ],
        out_specs=pl.BlockSpec((1, BM, D), lambda i, m, e: (i, 0, 0)),
    )
    return pl.pallas_call(
        _ffn_tile,
        grid_spec=grid_spec,
        out_shape=jax.ShapeDtypeStruct((NT, BM, D), jnp.float32),
        compiler_params=pltpu.CompilerParams(
            dimension_semantics=("arbitrary",)),
    )(tile_m, tile_e, x_sorted, w_up, w_down)


def kernel(hidden_states, top_k_index, top_k_weights, W_up, W_down):
    flat_e = top_k_index.astype(jnp.int32).T.reshape(A)
    pos, gidx, tile_m, tile_e = _routing_plan(flat_e)
    order = jnp.zeros((A,), jnp.int32).at[pos].set(
        jnp.arange(A, dtype=jnp.int32))
    x_sorted = jnp.take(hidden_states, order % T, axis=0)
    o_tiles = _grouped_ffn(x_sorted, W_up.astype(jnp.bfloat16),
                           W_down.astype(jnp.bfloat16), tile_m, tile_e)
    return o_tiles.reshape(NT * BM, D)[:T]


# speedup vs baseline: 1.4012x; 1.4012x over previous
import functools

import jax
import jax.numpy as jnp
from jax import lax
from jax.experimental import pallas as pl
from jax.experimental.pallas import tpu as pltpu
from jax.experimental.pallas import tpu_sc as plsc

E = 8
TOPK = 2
T = 2048
D = 1024
F = 1024
A = T * TOPK
BM = 256
NB = A // BM
NT = NB + E - 1
NTP = 128

NC = 2
NS = 16
NW = NC * NS
TPW = T // NW


def _lane_cumsum(x):
    n = x.shape[-1]
    k = 1
    while k < n:
        shifted = jnp.concatenate(
            [jnp.zeros(x.shape[:-1] + (k,), x.dtype), x[..., :n - k]],
            axis=-1)
        x = x + shifted
        k *= 2
    return x


def _sublane_excl_cumsum(x):
    incl = x
    k = 1
    while k < E:
        shifted = jnp.concatenate(
            [jnp.zeros((k, 1), x.dtype), incl[:E - k]], axis=0)
        incl = incl + shifted
        k *= 2
    return incl - x


def _prep_body(fe_ref, pos_ref, gidx_ref, tmap_ref):
    fe = fe_ref[...].reshape(1, A)
    eids = lax.broadcasted_iota(jnp.int32, (E, 1), 0)
    oh = (fe == eids).astype(jnp.int32)
    cs = _lane_cumsum(oh)
    counts = cs[:, A - 1:A]
    offs = _sublane_excl_cumsum(counts)
    ends = offs + counts

    rank = jnp.sum(oh * cs, axis=0, keepdims=True) - 1
    start_of = jnp.sum(oh * offs, axis=0, keepdims=True)
    pos = start_of + rank
    pos_ref[...] = pos.reshape(A)

    nonempty = (counts > 0).astype(jnp.int32)
    first_blk = offs // BM
    nb_e = nonempty * ((ends - 1) // BM - first_blk + 1)
    cum_t = _sublane_excl_cumsum(nb_e)
    cum_ti = cum_t + nb_e
    total = jnp.sum(nb_e, axis=0, keepdims=True)

    slot = lax.broadcasted_iota(jnp.int32, (1, NTP), 1)
    e_of = jnp.sum((slot >= cum_ti).astype(jnp.int32), axis=0, keepdims=True)
    e_of = jnp.minimum(e_of, E - 1)
    e_mask = (e_of == eids).astype(jnp.int32)
    adj = jnp.sum(e_mask * (first_blk - cum_t), axis=0, keepdims=True)
    valid = slot < total
    m_of = jnp.where(valid, slot + adj, NB - 1)
    e_t = jnp.where(valid, e_of, 0)
    tmap = jnp.concatenate(
        [m_of, e_t, jnp.zeros((6, NTP), jnp.int32)], axis=0)
    tmap_ref[...] = tmap

    ep = jnp.sum((pos >= ends).astype(jnp.int32), axis=0, keepdims=True)
    ep_mask = (ep == eids).astype(jnp.int32)
    adj2 = jnp.sum(ep_mask * (cum_t - first_blk), axis=0, keepdims=True)
    tile_of = adj2 + pos // BM
    gidx = tile_of * BM + (pos & (BM - 1))
    gidx_ref[...] = gidx.reshape(A)


def _routing_prep(flat_e):
    return pl.pallas_call(
        _prep_body,
        out_shape=(
            jax.ShapeDtypeStruct((A,), jnp.int32),
            jax.ShapeDtypeStruct((A,), jnp.int32),
            jax.ShapeDtypeStruct((E, NTP), jnp.int32),
        ),
    )(flat_e)


def _dispatch(hidden_states, pos):
    mesh = plsc.VectorSubcoreMesh(core_axis_name="c", subcore_axis_name="s")

    @functools.partial(
        pl.kernel, mesh=mesh,
        out_type=jax.ShapeDtypeStruct((A, D), jnp.float32),
        scratch_types=[
            pltpu.VMEM((TPW,), jnp.int32),
            pltpu.VMEM((TPW,), jnp.int32),
            pltpu.VMEM((TPW, D), jnp.float32),
            pltpu.SemaphoreType.DMA,
            pltpu.SemaphoreType.DMA,
        ],
    )
    def k(hs_hbm, pos_hbm, xs_hbm, idx0, idx1, rows, sem0, sem1):
        wid = lax.axis_index("s") * NC + lax.axis_index("c")
        base = wid * TPW
        pltpu.sync_copy(pos_hbm.at[pl.ds(base, TPW)], idx0)
        pltpu.sync_copy(pos_hbm.at[pl.ds(T + base, TPW)], idx1)
        pltpu.sync_copy(hs_hbm.at[pl.ds(base, TPW)], rows)
        c0 = pltpu.async_copy(rows, xs_hbm.at[idx0], sem0)
        c1 = pltpu.async_copy(rows, xs_hbm.at[idx1], sem1)
        c0.wait()
        c1.wait()

    return k(hidden_states, pos)


def _ffn_tile(tmap_ref, x_ref, wu_ref, wd_ref, o_ref):
    h = jnp.dot(x_ref[...].astype(jnp.bfloat16),
                wu_ref[0].astype(jnp.bfloat16),
                preferred_element_type=jnp.float32)
    a = jnp.maximum(h, 0.0)
    a2 = (a * a).astype(jnp.bfloat16)
    o_ref[0] = jnp.dot(a2, wd_ref[0].astype(jnp.bfloat16),
                       preferred_element_type=jnp.float32)


def _grouped_ffn(x_sorted, w_up, w_down, tmap):
    grid_spec = pltpu.PrefetchScalarGridSpec(
        num_scalar_prefetch=1,
        grid=(NT,),
        in_specs=[
            pl.BlockSpec((BM, D), lambda i, t: (t[0, i], 0)),
            pl.BlockSpec((1, D, F), lambda i, t: (t[1, i], 0, 0)),
            pl.BlockSpec((1, F, D), lambda i, t: (t[1, i], 0, 0)),
        ],
        out_specs=pl.BlockSpec((1, BM, D), lambda i, t: (i, 0, 0)),
    )
    return pl.pallas_call(
        _ffn_tile,
        grid_spec=grid_spec,
        out_shape=jax.ShapeDtypeStruct((NT, BM, D), jnp.float32),
        compiler_params=pltpu.CompilerParams(
            dimension_semantics=("arbitrary",)),
    )(tmap, x_sorted, w_up, w_down)


def _combine(o_flat, gidx, w_bc):
    mesh = plsc.VectorSubcoreMesh(core_axis_name="c", subcore_axis_name="s")
    CT = 16
    NCH = TPW // CT
    NG = D // 16

    @functools.partial(
        pl.kernel, mesh=mesh,
        out_type=jax.ShapeDtypeStruct((T, D), jnp.float32),
        scratch_types=[
            pltpu.VMEM((TPW,), jnp.int32),
            pltpu.VMEM((TPW,), jnp.int32),
            pltpu.VMEM((TPW, 16), jnp.float32),
            pltpu.VMEM((TPW, 16), jnp.float32),
            pltpu.VMEM((CT, D), jnp.float32),
            pltpu.VMEM((CT, D), jnp.float32),
            pltpu.VMEM((CT, D), jnp.float32),
            pltpu.VMEM((CT, D), jnp.float32),
            pltpu.VMEM((CT, D), jnp.float32),
            pltpu.VMEM((CT, D), jnp.float32),
            pltpu.SemaphoreType.DMA,
            pltpu.SemaphoreType.DMA,
            pltpu.SemaphoreType.DMA,
            pltpu.SemaphoreType.DMA,
        ],
    )
    def k(of_hbm, gidx_hbm, wbc_hbm, out_hbm, idxa, idxb, wva, wvb,
          buf_a0, buf_b0, buf_a1, buf_b1, outv0, outv1,
          gsem0, gsem1, osem0, osem1):
        wid = lax.axis_index("s") * NC + lax.axis_index("c")
        base = wid * TPW
        pltpu.sync_copy(gidx_hbm.at[pl.ds(base, TPW)], idxa)
        pltpu.sync_copy(gidx_hbm.at[pl.ds(T + base, TPW)], idxb)
        pltpu.sync_copy(wbc_hbm.at[pl.ds(base, TPW)], wva)
        pltpu.sync_copy(wbc_hbm.at[pl.ds(T + base, TPW)], wvb)
        bufs = ((buf_a0, buf_b0, outv0, gsem0, osem0),
                (buf_a1, buf_b1, outv1, gsem1, osem1))
        gather_pending = [None, None]
        out_pending = [None, None]

        def fire(c):
            a, b, _, gsem, _ = bufs[c % 2]
            ga = pltpu.async_copy(
                of_hbm.at[idxa.at[pl.ds(c * CT, CT)]], a, gsem)
            gb = pltpu.async_copy(
                of_hbm.at[idxb.at[pl.ds(c * CT, CT)]], b, gsem)
            gather_pending[c % 2] = (ga, gb)

        fire(0)
        for c in range(NCH):
            if c + 1 < NCH:
                fire(c + 1)
            a, b, outv, _, osem = bufs[c % 2]
            ga, gb = gather_pending[c % 2]
            ga.wait()
            gb.wait()
            if out_pending[c % 2] is not None:
                out_pending[c % 2].wait()
                out_pending[c % 2] = None
            for j in range(CT):
                w0 = wva[c * CT + j]
                w1 = wvb[c * CT + j]

                def add_grp(g, _):
                    for u in range(8):
                        sl = pl.ds((g * 8 + u) * 16, 16)
                        outv[j, sl] = w0 * a[j, sl] + w1 * b[j, sl]
                    return 0

                lax.fori_loop(0, NG // 8, add_grp, 0)
            out_pending[c % 2] = pltpu.async_copy(
                outv, out_hbm.at[pl.ds(base + c * CT, CT)], osem)
        for p in range(2):
            if out_pending[p] is not None:
                out_pending[p].wait()

    return k(o_flat, gidx, w_bc)


def kernel(hidden_states, top_k_index, top_k_weights, W_up, W_down):
    flat_e = top_k_index.astype(jnp.int32).T.reshape(A)
    w_bc = jnp.broadcast_to(
        top_k_weights.T.reshape(A, 1), (A, 16)).astype(jnp.float32)
    pos, gidx, tmap = _routing_prep(flat_e)
    x_sorted = _dispatch(hidden_states, pos)
    o_tiles = _grouped_ffn(x_sorted, W_up, W_down, tmap)
    return _combine(o_tiles.reshape(NT * BM, D), gidx, w_bc)


# speedup vs baseline: 1.4645x; 1.0452x over previous
import functools

import jax
import jax.numpy as jnp
from jax import lax
from jax.experimental import pallas as pl
from jax.experimental.pallas import tpu as pltpu
from jax.experimental.pallas import tpu_sc as plsc

E = 8
TOPK = 2
T = 2048
D = 1024
F = 1024
A = T * TOPK
BM = 512
NB = A // BM
NT = NB + E - 1
NTP = 128

NC = 2
NS = 16
NW = NC * NS
TPW = T // NW


def _lane_cumsum(x):
    n = x.shape[-1]
    k = 1
    while k < n:
        shifted = jnp.concatenate(
            [jnp.zeros(x.shape[:-1] + (k,), x.dtype), x[..., :n - k]],
            axis=-1)
        x = x + shifted
        k *= 2
    return x


def _sublane_excl_cumsum(x):
    incl = x
    k = 1
    while k < E:
        shifted = jnp.concatenate(
            [jnp.zeros((k, 1), x.dtype), incl[:E - k]], axis=0)
        incl = incl + shifted
        k *= 2
    return incl - x


def _prep_body(fe_ref, pos_ref, gidx_ref, tmap_ref):
    fe = fe_ref[...].reshape(1, A)
    eids = lax.broadcasted_iota(jnp.int32, (E, 1), 0)
    oh = (fe == eids).astype(jnp.int32)
    cs = _lane_cumsum(oh)
    counts = cs[:, A - 1:A]
    offs = _sublane_excl_cumsum(counts)
    ends = offs + counts

    rank = jnp.sum(oh * cs, axis=0, keepdims=True) - 1
    start_of = jnp.sum(oh * offs, axis=0, keepdims=True)
    pos = start_of + rank
    pos_ref[...] = pos.reshape(A)

    nonempty = (counts > 0).astype(jnp.int32)
    first_blk = offs // BM
    nb_e = nonempty * ((ends - 1) // BM - first_blk + 1)
    cum_t = _sublane_excl_cumsum(nb_e)
    cum_ti = cum_t + nb_e
    total = jnp.sum(nb_e, axis=0, keepdims=True)

    slot = lax.broadcasted_iota(jnp.int32, (1, NTP), 1)
    e_of = jnp.sum((slot >= cum_ti).astype(jnp.int32), axis=0, keepdims=True)
    e_of = jnp.minimum(e_of, E - 1)
    e_mask = (e_of == eids).astype(jnp.int32)
    adj = jnp.sum(e_mask * (first_blk - cum_t), axis=0, keepdims=True)
    valid = slot < total
    m_of = jnp.where(valid, slot + adj, NB - 1)
    e_t = jnp.where(valid, e_of, 0)
    tmap = jnp.concatenate(
        [m_of, e_t, jnp.zeros((6, NTP), jnp.int32)], axis=0)
    tmap_ref[...] = tmap

    ep = jnp.sum((pos >= ends).astype(jnp.int32), axis=0, keepdims=True)
    ep_mask = (ep == eids).astype(jnp.int32)
    adj2 = jnp.sum(ep_mask * (cum_t - first_blk), axis=0, keepdims=True)
    tile_of = adj2 + pos // BM
    gidx = tile_of * BM + (pos & (BM - 1))
    gidx_ref[...] = gidx.reshape(A)


def _routing_prep(flat_e):
    return pl.pallas_call(
        _prep_body,
        out_shape=(
            jax.ShapeDtypeStruct((A,), jnp.int32),
            jax.ShapeDtypeStruct((A,), jnp.int32),
            jax.ShapeDtypeStruct((E, NTP), jnp.int32),
        ),
    )(flat_e)


def _dispatch(hidden_states, pos):
    mesh = plsc.VectorSubcoreMesh(core_axis_name="c", subcore_axis_name="s")

    @functools.partial(
        pl.kernel, mesh=mesh,
        out_type=jax.ShapeDtypeStruct((A, D), jnp.float32),
        scratch_types=[
            pltpu.VMEM((TPW,), jnp.int32),
            pltpu.VMEM((TPW,), jnp.int32),
            pltpu.VMEM((TPW, D), jnp.float32),
            pltpu.SemaphoreType.DMA,
            pltpu.SemaphoreType.DMA,
        ],
    )
    def k(hs_hbm, pos_hbm, xs_hbm, idx0, idx1, rows, sem0, sem1):
        wid = lax.axis_index("s") * NC + lax.axis_index("c")
        base = wid * TPW
        pltpu.sync_copy(pos_hbm.at[pl.ds(base, TPW)], idx0)
        pltpu.sync_copy(pos_hbm.at[pl.ds(T + base, TPW)], idx1)
        pltpu.sync_copy(hs_hbm.at[pl.ds(base, TPW)], rows)
        c0 = pltpu.async_copy(rows, xs_hbm.at[idx0], sem0)
        c1 = pltpu.async_copy(rows, xs_hbm.at[idx1], sem1)
        c0.wait()
        c1.wait()

    return k(hidden_states, pos)


def _ffn_tile(tmap_ref, x_ref, wu_ref, wd_ref, o_ref):
    h = jnp.dot(x_ref[...].astype(jnp.bfloat16),
                wu_ref[0].astype(jnp.bfloat16),
                preferred_element_type=jnp.float32)
    a = jnp.maximum(h, 0.0)
    a2 = (a * a).astype(jnp.bfloat16)
    o_ref[0] = jnp.dot(a2, wd_ref[0].astype(jnp.bfloat16),
                       preferred_element_type=jnp.float32)


def _grouped_ffn(x_sorted, w_up, w_down, tmap):
    grid_spec = pltpu.PrefetchScalarGridSpec(
        num_scalar_prefetch=1,
        grid=(NT,),
        in_specs=[
            pl.BlockSpec((BM, D), lambda i, t: (t[0, i], 0)),
            pl.BlockSpec((1, D, F), lambda i, t: (t[1, i], 0, 0)),
            pl.BlockSpec((1, F, D), lambda i, t: (t[1, i], 0, 0)),
        ],
        out_specs=pl.BlockSpec((1, BM, D), lambda i, t: (i, 0, 0)),
    )
    return pl.pallas_call(
        _ffn_tile,
        grid_spec=grid_spec,
        out_shape=jax.ShapeDtypeStruct((NT, BM, D), jnp.float32),
        compiler_params=pltpu.CompilerParams(
            dimension_semantics=("arbitrary",)),
    )(tmap, x_sorted, w_up, w_down)


def _combine(o_flat, gidx, w_bc):
    mesh = plsc.VectorSubcoreMesh(core_axis_name="c", subcore_axis_name="s")
    CT = 16
    NCH = TPW // CT
    NG = D // 16

    @functools.partial(
        pl.kernel, mesh=mesh,
        out_type=jax.ShapeDtypeStruct((T, D), jnp.float32),
        scratch_types=[
            pltpu.VMEM((TPW,), jnp.int32),
            pltpu.VMEM((TPW,), jnp.int32),
            pltpu.VMEM((TPW, 16), jnp.float32),
            pltpu.VMEM((TPW, 16), jnp.float32),
            pltpu.VMEM((CT, D), jnp.float32),
            pltpu.VMEM((CT, D), jnp.float32),
            pltpu.VMEM((CT, D), jnp.float32),
            pltpu.VMEM((CT, D), jnp.float32),
            pltpu.VMEM((CT, D), jnp.float32),
            pltpu.VMEM((CT, D), jnp.float32),
            pltpu.SemaphoreType.DMA,
            pltpu.SemaphoreType.DMA,
            pltpu.SemaphoreType.DMA,
            pltpu.SemaphoreType.DMA,
        ],
    )
    def k(of_hbm, gidx_hbm, wbc_hbm, out_hbm, idxa, idxb, wva, wvb,
          buf_a0, buf_b0, buf_a1, buf_b1, outv0, outv1,
          gsem0, gsem1, osem0, osem1):
        wid = lax.axis_index("s") * NC + lax.axis_index("c")
        base = wid * TPW
        pltpu.sync_copy(gidx_hbm.at[pl.ds(base, TPW)], idxa)
        pltpu.sync_copy(gidx_hbm.at[pl.ds(T + base, TPW)], idxb)
        pltpu.sync_copy(wbc_hbm.at[pl.ds(base, TPW)], wva)
        pltpu.sync_copy(wbc_hbm.at[pl.ds(T + base, TPW)], wvb)
        bufs = ((buf_a0, buf_b0, outv0, gsem0, osem0),
                (buf_a1, buf_b1, outv1, gsem1, osem1))
        gather_pending = [None, None]
        out_pending = [None, None]

        def fire(c):
            a, b, _, gsem, _ = bufs[c % 2]
            ga = pltpu.async_copy(
                of_hbm.at[idxa.at[pl.ds(c * CT, CT)]], a, gsem)
            gb = pltpu.async_copy(
                of_hbm.at[idxb.at[pl.ds(c * CT, CT)]], b, gsem)
            gather_pending[c % 2] = (ga, gb)

        fire(0)
        for c in range(NCH):
            if c + 1 < NCH:
                fire(c + 1)
            a, b, outv, _, osem = bufs[c % 2]
            ga, gb = gather_pending[c % 2]
            ga.wait()
            gb.wait()
            if out_pending[c % 2] is not None:
                out_pending[c % 2].wait()
                out_pending[c % 2] = None
            for j in range(CT):
                w0 = wva[c * CT + j]
                w1 = wvb[c * CT + j]

                def add_grp(g, _):
                    for u in range(8):
                        sl = pl.ds((g * 8 + u) * 16, 16)
                        outv[j, sl] = w0 * a[j, sl] + w1 * b[j, sl]
                    return 0

                lax.fori_loop(0, NG // 8, add_grp, 0)
            out_pending[c % 2] = pltpu.async_copy(
                outv, out_hbm.at[pl.ds(base + c * CT, CT)], osem)
        for p in range(2):
            if out_pending[p] is not None:
                out_pending[p].wait()

    return k(o_flat, gidx, w_bc)


def kernel(hidden_states, top_k_index, top_k_weights, W_up, W_down):
    flat_e = top_k_index.astype(jnp.int32).T.reshape(A)
    w_bc = jnp.broadcast_to(
        top_k_weights.T.reshape(A, 1), (A, 16)).astype(jnp.float32)
    pos, gidx, tmap = _routing_prep(flat_e)
    x_sorted = _dispatch(hidden_states, pos)
    o_tiles = _grouped_ffn(x_sorted, W_up, W_down, tmap)
    return _combine(o_tiles.reshape(NT * BM, D), gidx, w_bc)
